# baseline (speedup 1.0000x reference)
import jax
import jax.numpy as jnp
import numpy as np
from jax.experimental import pallas as pl
from jax.experimental.pallas import tpu as pltpu

B, T, D, H, DH, FF = 1, 2048, 1024, 1024 // 64, 64, 2816
EPS = 1e-6
THETA = 10000.0
BT = 512
BF = 512
BQ = 512
BKV = 512
HD = DH // 2


def _qkv_body(x_ref, ln1_ref, wq_ref, wk_ref, wv_ref, b_ref,
              cos_ref, sin_ref, qe_ref, qo_ref, k_ref, v_ref):
    bf16 = jnp.bfloat16
    x = x_ref[...]
    h = x * jax.lax.rsqrt(jnp.mean(x * x, axis=-1, keepdims=True) + EPS)
    h = h * ln1_ref[...]
    hb = h.astype(bf16)
    hq = (h * np.float32(np.log2(np.e) / np.sqrt(DH))).astype(bf16)
    q = (jnp.dot(hq, wq_ref[...].astype(bf16),
                 preferred_element_type=jnp.float32)
         + b_ref[:, :D]).astype(bf16)
    k = (jnp.dot(hb, wk_ref[...].astype(bf16),
                 preferred_element_type=jnp.float32)
         + b_ref[:, D:2 * D]).astype(bf16)
    v = jnp.dot(hb, wv_ref[...].astype(bf16),
                preferred_element_type=jnp.float32) + b_ref[:, 2 * D:]
    cos = cos_ref[...]
    sin = sin_ref[...]
    lane = jax.lax.broadcasted_iota(jnp.int32, (BT, D), 1)
    first = (lane % DH) < HD

    def rope(t):
        rot = jnp.where(first, -pltpu.roll(t, D - HD, 1),
                        pltpu.roll(t, HD, 1))
        return t * cos + rot * sin

    qr = rope(q)
    zero = jnp.zeros((), bf16)
    even = (lane % (2 * DH)) < DH
    qe_ref[...] = jnp.where(even, qr, zero)
    qo_ref[...] = jnp.where(even, zero, qr)
    k_ref[...] = rope(k)
    v_ref[...] = v.astype(bf16)


def _flash_body(qe_ref, qo_ref, k_ref, v_ref, o_ref):
    iq = pl.program_id(1)
    qq = jnp.concatenate([qe_ref[...], qo_ref[...]], axis=0)
    tri2 = (jax.lax.broadcasted_iota(jnp.int32, (2 * BQ, BKV), 0) % BQ >=
            jax.lax.broadcasted_iota(jnp.int32, (2 * BQ, BKV), 1))

    def pmat(kc):
        s = jax.lax.dot_general(qq, kc, (((1,), (1,)), ((), ())),
                                preferred_element_type=jnp.float32)
        return jnp.exp2(jnp.minimum(s, 100.0))

    def pv(p, vc):
        return jnp.dot(p.astype(jnp.bfloat16), vc,
                       preferred_element_type=jnp.float32)

    def rsum(p):
        return jnp.sum(p, axis=1, keepdims=True)

    def branch(n):
        def f():
            p = jnp.where(tri2, pmat(k_ref[pl.ds(n * BQ, BKV), :]), 0.0)
            a = pv(p, v_ref[pl.ds(n * BQ, BKV), :])
            l = rsum(p)
            if n:
                p2 = pmat(k_ref[pl.ds(0, n * BKV), :])
                a = a + pv(p2, v_ref[pl.ds(0, n * BKV), :])
                l = l + rsum(p2)
            out0 = a[:BQ, :DH] / l[:BQ]
            out1 = a[BQ:, DH:] / l[BQ:]
            o_ref[...] = jnp.concatenate([out0, out1],
                                         axis=1).astype(jnp.bfloat16)
        return f

    jax.lax.switch(iq, [branch(n) for n in range(T // BQ)])


def _ffn_body(attn_ref, x_ref, wo_ref, ln2_ref, wg_ref, wu_ref, wd_ref,
              o_ref):
    bf16 = jnp.bfloat16
    x2 = x_ref[...] + jnp.dot(attn_ref[...], wo_ref[...].astype(bf16),
                              preferred_element_type=jnp.float32)
    h2 = x2 * jax.lax.rsqrt(jnp.mean(x2 * x2, axis=-1, keepdims=True) + EPS)
    h2 = (h2 * ln2_ref[...]).astype(bf16)
    g = jnp.dot(h2, wg_ref[...].astype(bf16),
                preferred_element_type=jnp.float32)
    u = jnp.dot(h2, wu_ref[...].astype(bf16),
                preferred_element_type=jnp.float32)
    mlp = (g * jax.nn.sigmoid(g) * u).astype(bf16)
    o_ref[...] = x2 + jnp.dot(mlp, wd_ref[...].astype(bf16),
                              preferred_element_type=jnp.float32)


def kernel(hidden_states, position_ids, Wq, bq, Wk, bk, Wv, bv, Wo,
           Wg, Wu, Wd, ln1, ln2):
    f32, bf16 = jnp.float32, jnp.bfloat16
    x = hidden_states[0]
    pos = position_ids[0].astype(f32)
    inv_freq = 1.0 / (THETA ** (jnp.arange(0, DH, 2, dtype=f32) / DH))
    ang = pos[:, None] * inv_freq[None, :]
    cosf = jnp.tile(jnp.concatenate([jnp.cos(ang)] * 2, -1),
                    (1, H)).astype(bf16)
    sinf = jnp.tile(jnp.concatenate([jnp.sin(ang)] * 2, -1),
                    (1, H)).astype(bf16)
    scale = np.log2(np.e) / np.sqrt(DH)
    ball = jnp.concatenate([bq * scale, bk, bv])[None, :]

    full = lambda shape: pl.BlockSpec(shape, lambda i: (0,) * len(shape))
    rows = lambda r, w: pl.BlockSpec((r, w), lambda i: (i, 0))

    qe, qo, k, v = pl.pallas_call(
        _qkv_body,
        grid=(T // BT,),
        in_specs=[rows(BT, D), full((1, D)), full((D, D)), full((D, D)),
                  full((D, D)), full((1, 3 * D)), rows(BT, D), rows(BT, D)],
        out_specs=[rows(BT, D)] * 4,
        out_shape=[jax.ShapeDtypeStruct((T, D), bf16)] * 4,
        compiler_params=pltpu.CompilerParams(
            dimension_semantics=("parallel",)),
    )(x, ln1[None, :], Wq, Wk, Wv, ball, cosf, sinf)

    attn = pl.pallas_call(
        _flash_body,
        grid=(H // 2, T // BQ),
        in_specs=[pl.BlockSpec((BQ, 2 * DH), lambda h, i: (i, h)),
                  pl.BlockSpec((BQ, 2 * DH), lambda h, i: (i, h)),
                  pl.BlockSpec((T, 2 * DH), lambda h, i: (0, h)),
                  pl.BlockSpec((T, 2 * DH), lambda h, i: (0, h))],
        out_specs=pl.BlockSpec((BQ, 2 * DH), lambda h, i: (i, h)),
        out_shape=jax.ShapeDtypeStruct((T, D), bf16),
        compiler_params=pltpu.CompilerParams(
            dimension_semantics=("parallel", "arbitrary")),
    )(qe, qo, k, v)

    out = pl.pallas_call(
        _ffn_body,
        grid=(T // BF,),
        in_specs=[rows(BF, D), rows(BF, D), full((D, D)), full((1, D)),
                  full((D, FF)), full((D, FF)), full((FF, D))],
        out_specs=rows(BF, D),
        out_shape=jax.ShapeDtypeStruct((T, D), f32),
        compiler_params=pltpu.CompilerParams(
            dimension_semantics=("parallel",)),
    )(attn, x, Wo, ln2[None, :], Wg, Wu, Wd)

    return out[None]


# speedup vs baseline: 1.1007x; 1.1007x over previous
import jax
import jax.numpy as jnp
import numpy as np
from jax.experimental import pallas as pl
from jax.experimental.pallas import tpu as pltpu

B, T, D, H, DH, FF = 1, 2048, 1024, 1024 // 64, 64, 2816
EPS = 1e-6
THETA = 10000.0
BT = 512
BF = 512
BQ = 512
BKV = 512
HD = DH // 2


def _qkv_body(x_ref, ln1_ref, wq_ref, wk_ref, wv_ref, b_ref,
              cos_ref, sin_ref, qe_ref, qo_ref, k_ref, v_ref):
    bf16 = jnp.bfloat16
    x = x_ref[...]
    h = x * jax.lax.rsqrt(jnp.mean(x * x, axis=-1, keepdims=True) + EPS)
    h = h * ln1_ref[...]
    hb = h.astype(bf16)
    hq = (h * np.float32(np.log2(np.e) / np.sqrt(DH))).astype(bf16)
    q = (jnp.dot(hq, wq_ref[...].astype(bf16),
                 preferred_element_type=jnp.float32)
         + b_ref[:, :D]).astype(bf16)
    k = (jnp.dot(hb, wk_ref[...].astype(bf16),
                 preferred_element_type=jnp.float32)
         + b_ref[:, D:2 * D]).astype(bf16)
    v = jnp.dot(hb, wv_ref[...].astype(bf16),
                preferred_element_type=jnp.float32) + b_ref[:, 2 * D:]
    cos = cos_ref[...]
    sin = sin_ref[...]
    lane = jax.lax.broadcasted_iota(jnp.int32, (BT, D), 1)
    first = (lane % DH) < HD

    def rope(t):
        rot = jnp.where(first, -pltpu.roll(t, D - HD, 1),
                        pltpu.roll(t, HD, 1))
        return t * cos + rot * sin

    qr = rope(q)
    zero = jnp.zeros((), bf16)
    even = (lane % (2 * DH)) < DH
    qe_ref[...] = jnp.where(even, qr, zero)
    qo_ref[...] = jnp.where(even, zero, qr)
    k_ref[...] = rope(k)
    v_ref[...] = v.astype(bf16)


def _flash_body(qe_ref, qo_ref, k_ref, v_ref, o_ref):
    iq = pl.program_id(1)
    tri = (jax.lax.broadcasted_iota(jnp.int32, (BQ, BKV), 0) >=
           jax.lax.broadcasted_iota(jnp.int32, (BQ, BKV), 1))
    qe = qe_ref[...]
    qo = qo_ref[...]

    def pmat(q2, kc):
        s = jax.lax.dot_general(q2, kc, (((1,), (1,)), ((), ())),
                                preferred_element_type=jnp.float32)
        return jnp.exp2(jnp.minimum(s, 100.0))

    def pv(p, vc):
        return jnp.dot(p.astype(jnp.bfloat16), vc,
                       preferred_element_type=jnp.float32)

    def rsum(p):
        return jnp.sum(p, axis=1, keepdims=True)

    def branch(n):
        def f():
            kd = k_ref[pl.ds(n * BQ, BKV), :]
            vd = v_ref[pl.ds(n * BQ, BKV), :]
            p0 = jnp.where(tri, pmat(qe, kd), 0.0)
            p1 = jnp.where(tri, pmat(qo, kd), 0.0)
            a0, a1 = pv(p0, vd), pv(p1, vd)
            l0, l1 = rsum(p0), rsum(p1)
            if n:
                kc = k_ref[pl.ds(0, n * BKV), :]
                vc = v_ref[pl.ds(0, n * BKV), :]
                p0 = pmat(qe, kc)
                p1 = pmat(qo, kc)
                a0, a1 = a0 + pv(p0, vc), a1 + pv(p1, vc)
                l0, l1 = l0 + rsum(p0), l1 + rsum(p1)
            out0 = a0[:, :DH] / l0
            out1 = a1[:, DH:] / l1
            o_ref[...] = jnp.concatenate([out0, out1],
                                         axis=1).astype(jnp.bfloat16)
        return f

    jax.lax.switch(iq, [branch(n) for n in range(T // BQ)])


def _ffn_body(attn_ref, x_ref, wo_ref, ln2_ref, wg_ref, wu_ref, wd_ref,
              o_ref):
    bf16 = jnp.bfloat16
    x2 = x_ref[...] + jnp.dot(attn_ref[...], wo_ref[...].astype(bf16),
                              preferred_element_type=jnp.float32)
    h2 = x2 * jax.lax.rsqrt(jnp.mean(x2 * x2, axis=-1, keepdims=True) + EPS)
    h2 = (h2 * ln2_ref[...]).astype(bf16)
    g = jnp.dot(h2, wg_ref[...].astype(bf16),
                preferred_element_type=jnp.float32)
    u = jnp.dot(h2, wu_ref[...].astype(bf16),
                preferred_element_type=jnp.float32)
    mlp = (g * jax.nn.sigmoid(g) * u).astype(bf16)
    o_ref[...] = x2 + jnp.dot(mlp, wd_ref[...].astype(bf16),
                              preferred_element_type=jnp.float32)


def kernel(hidden_states, position_ids, Wq, bq, Wk, bk, Wv, bv, Wo,
           Wg, Wu, Wd, ln1, ln2):
    f32, bf16 = jnp.float32, jnp.bfloat16
    x = hidden_states[0]
    pos = position_ids[0].astype(f32)
    inv_freq = 1.0 / (THETA ** (jnp.arange(0, DH, 2, dtype=f32) / DH))
    ang = pos[:, None] * inv_freq[None, :]
    cosf = jnp.tile(jnp.concatenate([jnp.cos(ang)] * 2, -1),
                    (1, H)).astype(bf16)
    sinf = jnp.tile(jnp.concatenate([jnp.sin(ang)] * 2, -1),
                    (1, H)).astype(bf16)
    scale = np.log2(np.e) / np.sqrt(DH)
    ball = jnp.concatenate([bq * scale, bk, bv])[None, :]

    full = lambda shape: pl.BlockSpec(shape, lambda i: (0,) * len(shape))
    rows = lambda r, w: pl.BlockSpec((r, w), lambda i: (i, 0))

    qe, qo, k, v = pl.pallas_call(
        _qkv_body,
        grid=(T // BT,),
        in_specs=[rows(BT, D), full((1, D)), full((D, D)), full((D, D)),
                  full((D, D)), full((1, 3 * D)), rows(BT, D), rows(BT, D)],
        out_specs=[rows(BT, D)] * 4,
        out_shape=[jax.ShapeDtypeStruct((T, D), bf16)] * 4,
        compiler_params=pltpu.CompilerParams(
            dimension_semantics=("parallel",)),
    )(x, ln1[None, :], Wq, Wk, Wv, ball, cosf, sinf)

    attn = pl.pallas_call(
        _flash_body,
        grid=(H // 2, T // BQ),
        in_specs=[pl.BlockSpec((BQ, 2 * DH), lambda h, i: (i, h)),
                  pl.BlockSpec((BQ, 2 * DH), lambda h, i: (i, h)),
                  pl.BlockSpec((T, 2 * DH), lambda h, i: (0, h)),
                  pl.BlockSpec((T, 2 * DH), lambda h, i: (0, h))],
        out_specs=pl.BlockSpec((BQ, 2 * DH), lambda h, i: (i, h)),
        out_shape=jax.ShapeDtypeStruct((T, D), bf16),
        compiler_params=pltpu.CompilerParams(
            dimension_semantics=("parallel", "parallel")),
    )(qe, qo, k, v)

    out = pl.pallas_call(
        _ffn_body,
        grid=(T // BF,),
        in_specs=[rows(BF, D), rows(BF, D), full((D, D)), full((1, D)),
                  full((D, FF)), full((D, FF)), full((FF, D))],
        out_specs=rows(BF, D),
        out_shape=jax.ShapeDtypeStruct((T, D), f32),
        compiler_params=pltpu.CompilerParams(
            dimension_semantics=("parallel",)),
    )(attn, x, Wo, ln2[None, :], Wg, Wu, Wd)

    return out[None]
